# Initial kernel scaffold; baseline (speedup 1.0000x reference)
import jax
import jax.numpy as jnp
from jax.experimental import pallas as pl


def kernel(x, edge_index, edge_type, w1, q1, k1, b1, w2, q2, k2, b2, lin_w, lin_b):
    raise NotImplementedError("write your pallas kernel here")


# speedup vs baseline: 14.9656x; 14.9656x over previous
import functools

import jax
import jax.numpy as jnp
from jax import lax
from jax.experimental import pallas as pl
from jax.experimental.pallas import tpu as pltpu
from jax.experimental.pallas import tpu_sc as plsc

N = 10000
E = 320000
C = 128
R = 16
NEG = 0.2
NC = 2
NS = 16
NW = NC * NS
CH = 128
NCH = -(-E // (NW * CH))
CHW = NCH * CH
EP = CHW * NW
NP = 10240
STR = NP // NS
DBITS = 14
DMASK = (1 << DBITS) - 1
BN = 1000

_mesh = plsc.VectorSubcoreMesh(core_axis_name="c", subcore_axis_name="s")
_f32 = jnp.float32
_i32 = jnp.int32
_sc_params = pltpu.CompilerParams(needs_layout_passes=False)


def _wproj_body(w1_ref, q1_ref, k1_ref, w2_ref, q2_ref, k2_ref,
                wq1_ref, wk1_ref, wq2_ref, wk2_ref):
    w1f = w1_ref[...].reshape(R * C, C)
    w2f = w2_ref[...].reshape(R * C, C)
    wq1_ref[...] = jnp.dot(w1f, q1_ref[...], preferred_element_type=_f32)
    wk1_ref[...] = jnp.dot(w1f, k1_ref[...], preferred_element_type=_f32)
    wq2_ref[...] = jnp.dot(w2f, q2_ref[...], preferred_element_type=_f32)
    wk2_ref[...] = jnp.dot(w2f, k2_ref[...], preferred_element_type=_f32)


_wproj = pl.pallas_call(
    _wproj_body,
    out_shape=[jax.ShapeDtypeStruct((R * C, 1), _f32)] * 4,
)


def _qk(xb, wq, wk):
    dn = (((1,), (1,)), ((), ()))
    qa = lax.dot_general(xb, wq, dn, preferred_element_type=_f32)
    ka = lax.dot_general(xb, wk, dn, preferred_element_type=_f32)
    return qa, ka


def _tc1_body(x_ref, wcat_ref, wq_ref, wk_ref, o_ref, qa_ref, ka_ref):
    xb = x_ref[...]
    o_ref[...] = jnp.dot(xb, wcat_ref[...], preferred_element_type=_f32)
    qa_ref[...], ka_ref[...] = _qk(xb, wq_ref[...], wk_ref[...])


_ospecs = dict(
    out_specs=[
        pl.BlockSpec((BN, R * C), lambda i: (i, 0)),
        pl.BlockSpec((BN, R), lambda i: (i, 0)),
        pl.BlockSpec((BN, R), lambda i: (i, 0)),
    ],
    out_shape=[
        jax.ShapeDtypeStruct((N, R * C), _f32),
        jax.ShapeDtypeStruct((N, R), _f32),
        jax.ShapeDtypeStruct((N, R), _f32),
    ],
)

_tc1 = pl.pallas_call(
    _tc1_body,
    grid=(N // BN,),
    in_specs=[
        pl.BlockSpec((BN, C), lambda i: (i, 0)),
        pl.BlockSpec((C, R * C), lambda i: (0, 0)),
        pl.BlockSpec((R, C), lambda i: (0, 0)),
        pl.BlockSpec((R, C), lambda i: (0, 0)),
    ],
    **_ospecs,
)


def _tcmid_body(p_ref, b_ref, wcat_ref, wq_ref, wk_ref,
                o_ref, qa_ref, ka_ref):
    h = jnp.maximum(p_ref[0] + p_ref[1] + b_ref[...], 0.0)
    o_ref[...] = jnp.dot(h, wcat_ref[...], preferred_element_type=_f32)
    qa_ref[...], ka_ref[...] = _qk(h, wq_ref[...], wk_ref[...])


_tcmid = pl.pallas_call(
    _tcmid_body,
    grid=(N // BN,),
    in_specs=[
        pl.BlockSpec((NC, BN, C), lambda i: (0, i, 0)),
        pl.BlockSpec((1, C), lambda i: (0, 0)),
        pl.BlockSpec((C, R * C), lambda i: (0, 0)),
        pl.BlockSpec((R, C), lambda i: (0, 0)),
        pl.BlockSpec((R, C), lambda i: (0, 0)),
    ],
    **_ospecs,
)


def _tcfin_body(p_ref, b_ref, linT_ref, lb_ref, o_ref):
    h = jnp.maximum(p_ref[0] + p_ref[1] + b_ref[...], 0.0)
    o_ref[...] = jnp.dot(h, linT_ref[...], preferred_element_type=_f32) + lb_ref[...]


_tcfin = pl.pallas_call(
    _tcfin_body,
    grid=(N // BN,),
    in_specs=[
        pl.BlockSpec((NC, BN, C), lambda i: (0, i, 0)),
        pl.BlockSpec((1, C), lambda i: (0, 0)),
        pl.BlockSpec((C, C), lambda i: (0, 0)),
        pl.BlockSpec((1, C), lambda i: (0, 0)),
    ],
    out_specs=pl.BlockSpec((BN, C), lambda i: (i, 0)),
    out_shape=jax.ShapeDtypeStruct((N, C), _f32),
)


@functools.partial(
    pl.kernel,
    out_type=[
        jax.ShapeDtypeStruct((EP,), _f32),
        jax.ShapeDtypeStruct((NC, NP), _f32),
        jax.ShapeDtypeStruct((EP,), _i32),
    ],
    mesh=_mesh,
    scratch_types=[
        pltpu.VMEM((CHW,), _i32),
        pltpu.VMEM((CHW,), _i32),
        pltpu.VMEM((CHW,), _i32),
        pltpu.VMEM((CHW,), _f32),
        pltpu.VMEM((CHW,), _i32),
        pltpu.VMEM((CH,), _i32),
        pltpu.VMEM((CH,), _i32),
        pltpu.VMEM((CH,), _f32),
        pltpu.VMEM((CH,), _f32),
        pltpu.VMEM((NP,), _f32),
        pltpu.VMEM((STR,), _f32),
        pltpu.VMEM((STR,), _f32),
        pltpu.VMEM_SHARED((NS, NP), _f32),
        pltpu.SemaphoreType.DMA,
        pltpu.SemaphoreType.DMA,
    ],
    compiler_params=_sc_params,
)
def _phase_a(qa_hbm, ka_hbm, src_hbm, dst_hbm, et_hbm,
             ev_out, den_out, pk_out,
             srcb, dstb, etb, evb, pkb, idxq, idxk, qsing, ksing,
             dloc, acc, tmp, stage, sem1, sem2):
    c = lax.axis_index("c")
    s = lax.axis_index("s")
    wid = s * NC + c
    base = wid * CHW
    pltpu.sync_copy(src_hbm.at[pl.ds(base, CHW)], srcb)
    pltpu.sync_copy(dst_hbm.at[pl.ds(base, CHW)], dstb)
    pltpu.sync_copy(et_hbm.at[pl.ds(base, CHW)], etb)
    zero16 = jnp.zeros((16,), _f32)

    def zb(i, _):
        dloc[pl.ds(i * 16, 16)] = zero16
        return 0

    lax.fori_loop(0, NP // 16, zb, 0, unroll=8)
    iota = lax.iota(_i32, 16)

    def chunk(ci, _):
        off = ci * CH
        for g in range(CH // 16):
            sl = pl.ds(g * 16, 16)
            gsl = pl.ds(off + g * 16, 16)
            etg = etb[gsl]
            dv = dstb[gsl]
            si = srcb[gsl] * R + etg
            idxq[sl] = dv * R + etg
            idxk[sl] = si
            pkb[gsl] = lax.shift_left(si, DBITS) | dv
        d1 = pltpu.async_copy(qa_hbm.at[idxq], qsing, sem1)
        d2 = pltpu.async_copy(ka_hbm.at[idxk], ksing, sem2)
        d1.wait()
        d2.wait()
        gbase = base + off
        for g in range(CH // 16):
            sl = pl.ds(g * 16, 16)
            gsl = pl.ds(off + g * 16, 16)
            a = qsing[sl] + ksing[sl]
            a = jnp.where(a > 0, a, a * NEG)
            ev = jnp.exp(a)
            eidx = gbase + g * 16 + iota
            ev = jnp.where(eidx < E, ev, 0.0)
            evb[gsl] = ev
            plsc.addupdate_scatter(dloc, [dstb[gsl]], ev)
        return 0

    lax.fori_loop(0, NCH, chunk, 0)
    pltpu.sync_copy(evb, ev_out.at[pl.ds(base, CHW)])
    pltpu.sync_copy(pkb, pk_out.at[pl.ds(base, CHW)])
    pltpu.sync_copy(dloc, stage.at[s])
    plsc.subcore_barrier()
    rb = s * STR
    pltpu.sync_copy(stage.at[0, pl.ds(rb, STR)], acc)

    def redj(j, _):
        pltpu.sync_copy(stage.at[j, pl.ds(rb, STR)], tmp)
        for g in range(STR // 16):
            sl = pl.ds(g * 16, 16)
            acc[sl] = acc[sl] + tmp[sl]
        return 0

    lax.fori_loop(1, NS, redj, 0)
    pltpu.sync_copy(acc, den_out.at[c, pl.ds(rb, STR)])


@functools.partial(
    pl.kernel,
    out_type=[
        jax.ShapeDtypeStruct((NC, NP, C), _f32),
        jax.ShapeDtypeStruct((EP,), _f32),
    ],
    mesh=_mesh,
    scratch_types=[
        pltpu.VMEM((CH,), _i32),
        pltpu.VMEM((CH,), _f32),
        pltpu.VMEM((CH,), _i32),
        pltpu.VMEM((CH,), _i32),
        pltpu.VMEM((CH, C), _f32),
        pltpu.VMEM((NP,), _f32),
        pltpu.VMEM((NP,), _f32),
        pltpu.VMEM_SHARED((NP, C), _f32),
        pltpu.SemaphoreType.DMA,
    ],
    compiler_params=_sc_params,
)
def _phase_b(ao_hbm, ev_hbm, den_hbm, pk_hbm,
             part_out, al_out,
             pkc, evc, idxb, didxb, rows, dfull, tmp, aggr, sem):
    c = lax.axis_index("c")
    s = lax.axis_index("s")
    wid = s * NC + c
    base = wid * CHW
    rb = s * STR
    zero16 = jnp.zeros((16,), _f32)

    def zrow(i, _):
        for j in range(C // 16):
            rows[i, pl.ds(j * 16, 16)] = zero16
        return 0

    lax.fori_loop(0, CH, zrow, 0)
    for p in range(STR // CH):
        pltpu.sync_copy(rows, aggr.at[pl.ds(rb + p * CH, CH)])

    pltpu.sync_copy(den_hbm.at[0], dfull)
    pltpu.sync_copy(den_hbm.at[1], tmp)

    def db(i, _):
        sl = pl.ds(i * 16, 16)
        dfull[sl] = dfull[sl] + tmp[sl]
        return 0

    lax.fori_loop(0, NP // 16, db, 0, unroll=8)
    plsc.subcore_barrier()

    def chunk(ci, _):
        off = ci * CH
        gb = base + off
        pltpu.sync_copy(pk_hbm.at[pl.ds(gb, CH)], pkc)
        pltpu.sync_copy(ev_hbm.at[pl.ds(gb, CH)], evc)
        for g in range(CH // 16):
            sl = pl.ds(g * 16, 16)
            pv = pkc[sl]
            idxb[sl] = lax.shift_right_logical(pv, DBITS)
            didxb[sl] = pv & DMASK
        dma = pltpu.async_copy(ao_hbm.at[idxb], rows, sem)
        for g in range(CH // 16):
            sl = pl.ds(g * 16, 16)
            den_g = plsc.load_gather(dfull, [didxb[sl]])
            evc[sl] = evc[sl] / (den_g + 1e-16)
        pltpu.sync_copy(evc, al_out.at[pl.ds(gb, CH)])
        dma.wait()

        def eb(i, _):
            ri = jnp.zeros((16,), _i32) + i
            siv = plsc.load_gather(evc, [ri])
            for j in range(C // 16):
                sl = pl.ds(j * 16, 16)
                rows[i, sl] = rows[i, sl] * siv
            return 0

        lax.fori_loop(0, CH, eb, 0)
        pltpu.sync_copy(rows, aggr.at[didxb], add=True)
        return 0

    lax.fori_loop(0, NCH, chunk, 0)
    plsc.subcore_barrier()
    pltpu.sync_copy(aggr.at[pl.ds(rb, STR)], part_out.at[c, pl.ds(rb, STR)])


def kernel(x, edge_index, edge_type, w1, q1, k1, b1, w2, q2, k2, b2,
           lin_w, lin_b):
    src = edge_index[0]
    dst = edge_index[1]
    pad = EP - E
    srcp = jnp.pad(src, (0, pad))
    dstp = jnp.pad(dst, (0, pad))
    etp = jnp.pad(edge_type, (0, pad))

    wcat1 = w1.transpose(1, 0, 2).reshape(C, R * C)
    wcat2 = w2.transpose(1, 0, 2).reshape(C, R * C)
    wq1, wk1, wq2, wk2 = _wproj(w1, q1, k1, w2, q2, k2)
    wq1 = wq1.reshape(R, C)
    wk1 = wk1.reshape(R, C)
    wq2 = wq2.reshape(R, C)
    wk2 = wk2.reshape(R, C)

    ao1, qa1, ka1 = _tc1(x, wcat1, wq1, wk1)
    ev1, den1, pk = _phase_a(qa1.reshape(N * R), ka1.reshape(N * R),
                             srcp, dstp, etp)
    part1, _ = _phase_b(ao1.reshape(N * R, C), ev1, den1, pk)
    ao2, qa2, ka2 = _tcmid(part1, b1.reshape(1, C), wcat2, wq2, wk2)
    ev2, den2, _ = _phase_a(qa2.reshape(N * R), ka2.reshape(N * R),
                            srcp, dstp, etp)
    part2, al2 = _phase_b(ao2.reshape(N * R, C), ev2, den2, pk)
    out = _tcfin(part2, b2.reshape(1, C), lin_w.T, lin_b.reshape(1, C))
    alpha2 = al2[:E].reshape(E, 1)
    return out, (edge_index, alpha2)


# speedup vs baseline: 16.6064x; 1.1096x over previous
import functools

import jax
import jax.numpy as jnp
from jax import lax
from jax.experimental import pallas as pl
from jax.experimental.pallas import tpu as pltpu
from jax.experimental.pallas import tpu_sc as plsc

N = 10000
E = 320000
C = 128
R = 16
NEG = 0.2
NC = 2
NS = 16
NW = NC * NS
CH = 128
NCH = -(-E // (NW * CH))
CHW = NCH * CH
EP = CHW * NW
CB = 64
NCB = CHW // CB
NP = 10240
STR = NP // NS
DBITS = 14
DMASK = (1 << DBITS) - 1
BN = 1000

_mesh = plsc.VectorSubcoreMesh(core_axis_name="c", subcore_axis_name="s")
_f32 = jnp.float32
_i32 = jnp.int32
_sc_params = pltpu.CompilerParams(needs_layout_passes=False)


def _wproj_body(w1_ref, q1_ref, k1_ref, w2_ref, q2_ref, k2_ref,
                wq1_ref, wk1_ref, wq2_ref, wk2_ref):
    w1f = w1_ref[...].reshape(R * C, C)
    w2f = w2_ref[...].reshape(R * C, C)
    wq1_ref[...] = jnp.dot(w1f, q1_ref[...], preferred_element_type=_f32)
    wk1_ref[...] = jnp.dot(w1f, k1_ref[...], preferred_element_type=_f32)
    wq2_ref[...] = jnp.dot(w2f, q2_ref[...], preferred_element_type=_f32)
    wk2_ref[...] = jnp.dot(w2f, k2_ref[...], preferred_element_type=_f32)


_wproj = pl.pallas_call(
    _wproj_body,
    out_shape=[jax.ShapeDtypeStruct((R * C, 1), _f32)] * 4,
)


def _qk(xb, wq, wk):
    dn = (((1,), (1,)), ((), ()))
    qa = lax.dot_general(xb, wq, dn, preferred_element_type=_f32)
    ka = lax.dot_general(xb, wk, dn, preferred_element_type=_f32)
    return qa, ka


def _tc1_body(x_ref, wcat_ref, wq_ref, wk_ref, o_ref, qa_ref, ka_ref):
    xb = x_ref[...]
    o_ref[...] = jnp.dot(xb, wcat_ref[...], preferred_element_type=_f32)
    qa_ref[...], ka_ref[...] = _qk(xb, wq_ref[...], wk_ref[...])


_ospecs = dict(
    out_specs=[
        pl.BlockSpec((BN, R * C), lambda i: (i, 0)),
        pl.BlockSpec((BN, R), lambda i: (i, 0)),
        pl.BlockSpec((BN, R), lambda i: (i, 0)),
    ],
    out_shape=[
        jax.ShapeDtypeStruct((N, R * C), _f32),
        jax.ShapeDtypeStruct((N, R), _f32),
        jax.ShapeDtypeStruct((N, R), _f32),
    ],
)

_tc1 = pl.pallas_call(
    _tc1_body,
    grid=(N // BN,),
    in_specs=[
        pl.BlockSpec((BN, C), lambda i: (i, 0)),
        pl.BlockSpec((C, R * C), lambda i: (0, 0)),
        pl.BlockSpec((R, C), lambda i: (0, 0)),
        pl.BlockSpec((R, C), lambda i: (0, 0)),
    ],
    **_ospecs,
)


def _tcmid_body(p_ref, b_ref, wcat_ref, wq_ref, wk_ref,
                o_ref, qa_ref, ka_ref):
    h = jnp.maximum(p_ref[0] + p_ref[1] + b_ref[...], 0.0)
    o_ref[...] = jnp.dot(h, wcat_ref[...], preferred_element_type=_f32)
    qa_ref[...], ka_ref[...] = _qk(h, wq_ref[...], wk_ref[...])


_tcmid = pl.pallas_call(
    _tcmid_body,
    grid=(N // BN,),
    in_specs=[
        pl.BlockSpec((NC, BN, C), lambda i: (0, i, 0)),
        pl.BlockSpec((1, C), lambda i: (0, 0)),
        pl.BlockSpec((C, R * C), lambda i: (0, 0)),
        pl.BlockSpec((R, C), lambda i: (0, 0)),
        pl.BlockSpec((R, C), lambda i: (0, 0)),
    ],
    **_ospecs,
)


def _tcfin_body(p_ref, b_ref, linT_ref, lb_ref, o_ref):
    h = jnp.maximum(p_ref[0] + p_ref[1] + b_ref[...], 0.0)
    o_ref[...] = jnp.dot(h, linT_ref[...], preferred_element_type=_f32) + lb_ref[...]


_tcfin = pl.pallas_call(
    _tcfin_body,
    grid=(N // BN,),
    in_specs=[
        pl.BlockSpec((NC, BN, C), lambda i: (0, i, 0)),
        pl.BlockSpec((1, C), lambda i: (0, 0)),
        pl.BlockSpec((C, C), lambda i: (0, 0)),
        pl.BlockSpec((1, C), lambda i: (0, 0)),
    ],
    out_specs=pl.BlockSpec((BN, C), lambda i: (i, 0)),
    out_shape=jax.ShapeDtypeStruct((N, C), _f32),
)


@functools.partial(
    pl.kernel,
    out_type=[
        jax.ShapeDtypeStruct((EP,), _f32),
        jax.ShapeDtypeStruct((NC, NP), _f32),
        jax.ShapeDtypeStruct((EP,), _i32),
    ],
    mesh=_mesh,
    scratch_types=[
        pltpu.VMEM((CHW,), _i32),
        pltpu.VMEM((CHW,), _i32),
        pltpu.VMEM((CHW,), _i32),
        pltpu.VMEM((CHW,), _f32),
        pltpu.VMEM((CHW,), _i32),
        pltpu.VMEM((CH,), _i32),
        pltpu.VMEM((CH,), _i32),
        pltpu.VMEM((CH,), _f32),
        pltpu.VMEM((CH,), _f32),
        pltpu.VMEM((NP,), _f32),
        pltpu.VMEM((STR,), _f32),
        pltpu.VMEM((STR,), _f32),
        pltpu.VMEM_SHARED((NS, NP), _f32),
        pltpu.SemaphoreType.DMA,
        pltpu.SemaphoreType.DMA,
    ],
    compiler_params=_sc_params,
)
def _phase_a(qa_hbm, ka_hbm, src_hbm, dst_hbm, et_hbm,
             ev_out, den_out, pk_out,
             srcb, dstb, etb, evb, pkb, idxq, idxk, qsing, ksing,
             dloc, acc, tmp, stage, sem1, sem2):
    c = lax.axis_index("c")
    s = lax.axis_index("s")
    wid = s * NC + c
    base = wid * CHW
    pltpu.sync_copy(src_hbm.at[pl.ds(base, CHW)], srcb)
    pltpu.sync_copy(dst_hbm.at[pl.ds(base, CHW)], dstb)
    pltpu.sync_copy(et_hbm.at[pl.ds(base, CHW)], etb)
    zero16 = jnp.zeros((16,), _f32)

    def zb(i, _):
        dloc[pl.ds(i * 16, 16)] = zero16
        return 0

    lax.fori_loop(0, NP // 16, zb, 0, unroll=8)
    iota = lax.iota(_i32, 16)

    def chunk(ci, _):
        off = ci * CH
        for g in range(CH // 16):
            sl = pl.ds(g * 16, 16)
            gsl = pl.ds(off + g * 16, 16)
            etg = etb[gsl]
            dv = dstb[gsl]
            si = srcb[gsl] * R + etg
            idxq[sl] = dv * R + etg
            idxk[sl] = si
            pkb[gsl] = lax.shift_left(si, DBITS) | dv
        d1 = pltpu.async_copy(qa_hbm.at[idxq], qsing, sem1)
        d2 = pltpu.async_copy(ka_hbm.at[idxk], ksing, sem2)
        d1.wait()
        d2.wait()
        gbase = base + off
        for g in range(CH // 16):
            sl = pl.ds(g * 16, 16)
            gsl = pl.ds(off + g * 16, 16)
            a = qsing[sl] + ksing[sl]
            a = jnp.where(a > 0, a, a * NEG)
            ev = jnp.exp(a)
            eidx = gbase + g * 16 + iota
            ev = jnp.where(eidx < E, ev, 0.0)
            evb[gsl] = ev
            plsc.addupdate_scatter(dloc, [dstb[gsl]], ev)
        return 0

    lax.fori_loop(0, NCH, chunk, 0)
    pltpu.sync_copy(evb, ev_out.at[pl.ds(base, CHW)])
    pltpu.sync_copy(pkb, pk_out.at[pl.ds(base, CHW)])
    pltpu.sync_copy(dloc, stage.at[s])
    plsc.subcore_barrier()
    rb = s * STR
    pltpu.sync_copy(stage.at[0, pl.ds(rb, STR)], acc)

    def redj(j, _):
        pltpu.sync_copy(stage.at[j, pl.ds(rb, STR)], tmp)
        for g in range(STR // 16):
            sl = pl.ds(g * 16, 16)
            acc[sl] = acc[sl] + tmp[sl]
        return 0

    lax.fori_loop(1, NS, redj, 0)
    pltpu.sync_copy(acc, den_out.at[c, pl.ds(rb, STR)])


@functools.partial(
    pl.kernel,
    out_type=[
        jax.ShapeDtypeStruct((NC, NP, C), _f32),
        jax.ShapeDtypeStruct((EP,), _f32),
    ],
    mesh=_mesh,
    scratch_types=[
        [pltpu.VMEM((CB,), _i32)] * 2,
        [pltpu.VMEM((CB,), _f32)] * 2,
        [pltpu.VMEM((CB,), _f32)] * 2,
        [pltpu.VMEM((CB,), _i32)] * 2,
        [pltpu.VMEM((CB,), _i32)] * 2,
        [pltpu.VMEM((CB, C), _f32)] * 2,
        pltpu.VMEM((NP,), _f32),
        pltpu.VMEM((NP,), _f32),
        pltpu.VMEM_SHARED((NP, C), _f32),
        [pltpu.SemaphoreType.DMA] * 2,
        [pltpu.SemaphoreType.DMA] * 2,
        [pltpu.SemaphoreType.DMA] * 2,
        [pltpu.SemaphoreType.DMA] * 2,
        [pltpu.SemaphoreType.DMA] * 2,
    ],
    compiler_params=_sc_params,
)
def _phase_b(ao_hbm, ev_hbm, den_hbm, pk_hbm,
             part_out, al_out,
             pkc, evc, alc, idxb, didxb, rows, dfull, tmp, aggr,
             semP, semE, semG, semS, semA):
    c = lax.axis_index("c")
    s = lax.axis_index("s")
    wid = s * NC + c
    base = wid * CHW
    rb = s * STR
    zero16 = jnp.zeros((16,), _f32)

    def zrow(i, _):
        for j in range(C // 16):
            rows[0][i, pl.ds(j * 16, 16)] = zero16
        return 0

    lax.fori_loop(0, CB, zrow, 0)
    for p in range(STR // CB):
        pltpu.sync_copy(rows[0], aggr.at[pl.ds(rb + p * CB, CB)])

    pltpu.sync_copy(den_hbm.at[0], dfull)
    pltpu.sync_copy(den_hbm.at[1], tmp)

    def db(i, _):
        sl = pl.ds(i * 16, 16)
        dfull[sl] = dfull[sl] + tmp[sl]
        return 0

    lax.fori_loop(0, NP // 16, db, 0, unroll=8)
    plsc.subcore_barrier()

    def load_meta(ci, b):
        gb = base + ci * CB
        pltpu.async_copy(pk_hbm.at[pl.ds(gb, CB)], pkc[b], semP[b])
        pltpu.async_copy(ev_hbm.at[pl.ds(gb, CB)], evc[b], semE[b])

    def wait_meta_pk(ci, b):
        gb = base + ci * CB
        pltpu.make_async_copy(pk_hbm.at[pl.ds(gb, CB)], pkc[b], semP[b]).wait()

    def wait_meta_ev(ci, b):
        gb = base + ci * CB
        pltpu.make_async_copy(ev_hbm.at[pl.ds(gb, CB)], evc[b], semE[b]).wait()

    load_meta(0, 0)
    load_meta(1, 1)

    def emit_front(cc, b, p):
        wait_meta_pk(cc, b)
        for g in range(CB // 16):
            sl = pl.ds(g * 16, 16)
            pv = pkc[b][sl]
            idxb[b][sl] = lax.shift_right_logical(pv, DBITS)
            didxb[b][sl] = pv & DMASK
        return pltpu.async_copy(ao_hbm.at[idxb[b]], rows[b], semG[b])

    def emit_back(cc, b, p, gdma):
        wait_meta_ev(cc, b)
        for g in range(CB // 16):
            sl = pl.ds(g * 16, 16)
            den_g = plsc.load_gather(dfull, [didxb[b][sl]])
            alc[b][sl] = evc[b][sl] / (den_g + 1e-16)
        pltpu.sync_copy(alc[b], al_out.at[pl.ds(base + cc * CB, CB)])

        @pl.when(cc + 2 < NCB)
        def _():
            load_meta(cc + 2, b)

        gdma.wait()

        def eb(i, _):
            ri = jnp.zeros((16,), _i32) + i
            siv = plsc.load_gather(alc[b], [ri])
            for j in range(C // 16):
                sl = pl.ds(j * 16, 16)
                rows[b][i, sl] = rows[b][i, sl] * siv
            return 0

        lax.fori_loop(0, CB, eb, 0, unroll=4)
        pltpu.sync_copy(rows[b], aggr.at[didxb[b]], add=True)

    def pair(p, _):
        c0 = 2 * p
        c1 = c0 + 1
        g0 = emit_front(c0, 0, p)
        g1 = emit_front(c1, 1, p)
        emit_back(c0, 0, p, g0)
        emit_back(c1, 1, p, g1)
        return 0

    lax.fori_loop(0, NCB // 2, pair, 0)
    plsc.subcore_barrier()
    pltpu.sync_copy(aggr.at[pl.ds(rb, STR)], part_out.at[c, pl.ds(rb, STR)])


def kernel(x, edge_index, edge_type, w1, q1, k1, b1, w2, q2, k2, b2,
           lin_w, lin_b):
    src = edge_index[0]
    dst = edge_index[1]
    pad = EP - E
    srcp = jnp.pad(src, (0, pad))
    dstp = jnp.pad(dst, (0, pad))
    etp = jnp.pad(edge_type, (0, pad))

    wcat1 = w1.transpose(1, 0, 2).reshape(C, R * C)
    wcat2 = w2.transpose(1, 0, 2).reshape(C, R * C)
    wq1, wk1, wq2, wk2 = _wproj(w1, q1, k1, w2, q2, k2)
    wq1 = wq1.reshape(R, C)
    wk1 = wk1.reshape(R, C)
    wq2 = wq2.reshape(R, C)
    wk2 = wk2.reshape(R, C)

    ao1, qa1, ka1 = _tc1(x, wcat1, wq1, wk1)
    ev1, den1, pk = _phase_a(qa1.reshape(N * R), ka1.reshape(N * R),
                             srcp, dstp, etp)
    part1, _ = _phase_b(ao1.reshape(N * R, C), ev1, den1, pk)
    ao2, qa2, ka2 = _tcmid(part1, b1.reshape(1, C), wcat2, wq2, wk2)
    ev2, den2, _ = _phase_a(qa2.reshape(N * R), ka2.reshape(N * R),
                            srcp, dstp, etp)
    part2, al2 = _phase_b(ao2.reshape(N * R, C), ev2, den2, pk)
    out = _tcfin(part2, b2.reshape(1, C), lin_w.T, lin_b.reshape(1, C))
    alpha2 = al2[:E].reshape(E, 1)
    return out, (edge_index, alpha2)


# speedup vs baseline: 18.9196x; 1.1393x over previous
import functools

import jax
import jax.numpy as jnp
from jax import lax
from jax.experimental import pallas as pl
from jax.experimental.pallas import tpu as pltpu
from jax.experimental.pallas import tpu_sc as plsc

N = 10000
E = 320000
C = 128
R = 16
NEG = 0.2
NC = 2
NS = 16
NW = NC * NS
CH = 128
NCH = -(-E // (NW * CH))
CHW = NCH * CH
EP = CHW * NW
CB = 64
NCB = CHW // CB
NP = 10240
STR = NP // NS
DBITS = 14
DMASK = (1 << DBITS) - 1
BN = 1000

_mesh = plsc.VectorSubcoreMesh(core_axis_name="c", subcore_axis_name="s")
_f32 = jnp.float32
_i32 = jnp.int32
_sc_params = pltpu.CompilerParams(needs_layout_passes=False)


def _wproj_body(w1_ref, q1_ref, k1_ref, w2_ref, q2_ref, k2_ref,
                wq1_ref, wk1_ref, wq2_ref, wk2_ref):
    w1f = w1_ref[...].reshape(R * C, C)
    w2f = w2_ref[...].reshape(R * C, C)
    wq1_ref[...] = jnp.dot(w1f, q1_ref[...], preferred_element_type=_f32)
    wk1_ref[...] = jnp.dot(w1f, k1_ref[...], preferred_element_type=_f32)
    wq2_ref[...] = jnp.dot(w2f, q2_ref[...], preferred_element_type=_f32)
    wk2_ref[...] = jnp.dot(w2f, k2_ref[...], preferred_element_type=_f32)


_wproj = pl.pallas_call(
    _wproj_body,
    out_shape=[jax.ShapeDtypeStruct((R * C, 1), _f32)] * 4,
)


def _qk(xb, wq, wk):
    dn = (((1,), (1,)), ((), ()))
    qa = lax.dot_general(xb, wq, dn, preferred_element_type=_f32)
    ka = lax.dot_general(xb, wk, dn, preferred_element_type=_f32)
    return qa, ka


def _tc1_body(x_ref, wcat_ref, wq_ref, wk_ref, o_ref, qa_ref, ka_ref):
    xb = x_ref[...]
    o_ref[...] = jnp.dot(xb, wcat_ref[...], preferred_element_type=_f32)
    qa_ref[...], ka_ref[...] = _qk(xb, wq_ref[...], wk_ref[...])


_ospecs = dict(
    out_specs=[
        pl.BlockSpec((BN, R * C), lambda i: (i, 0)),
        pl.BlockSpec((BN, R), lambda i: (i, 0)),
        pl.BlockSpec((BN, R), lambda i: (i, 0)),
    ],
    out_shape=[
        jax.ShapeDtypeStruct((N, R * C), _f32),
        jax.ShapeDtypeStruct((N, R), _f32),
        jax.ShapeDtypeStruct((N, R), _f32),
    ],
)

_tc1 = pl.pallas_call(
    _tc1_body,
    grid=(N // BN,),
    in_specs=[
        pl.BlockSpec((BN, C), lambda i: (i, 0)),
        pl.BlockSpec((C, R * C), lambda i: (0, 0)),
        pl.BlockSpec((R, C), lambda i: (0, 0)),
        pl.BlockSpec((R, C), lambda i: (0, 0)),
    ],
    **_ospecs,
)


def _tcmid_body(p_ref, b_ref, wcat_ref, wq_ref, wk_ref,
                o_ref, qa_ref, ka_ref):
    h = jnp.maximum(p_ref[0] + p_ref[1] + b_ref[...], 0.0)
    o_ref[...] = jnp.dot(h, wcat_ref[...], preferred_element_type=_f32)
    qa_ref[...], ka_ref[...] = _qk(h, wq_ref[...], wk_ref[...])


_tcmid = pl.pallas_call(
    _tcmid_body,
    grid=(N // BN,),
    in_specs=[
        pl.BlockSpec((NC, BN, C), lambda i: (0, i, 0)),
        pl.BlockSpec((1, C), lambda i: (0, 0)),
        pl.BlockSpec((C, R * C), lambda i: (0, 0)),
        pl.BlockSpec((R, C), lambda i: (0, 0)),
        pl.BlockSpec((R, C), lambda i: (0, 0)),
    ],
    **_ospecs,
)


def _tcfin_body(p_ref, b_ref, linT_ref, lb_ref, o_ref):
    h = jnp.maximum(p_ref[0] + p_ref[1] + b_ref[...], 0.0)
    o_ref[...] = jnp.dot(h, linT_ref[...], preferred_element_type=_f32) + lb_ref[...]


_tcfin = pl.pallas_call(
    _tcfin_body,
    grid=(N // BN,),
    in_specs=[
        pl.BlockSpec((NC, BN, C), lambda i: (0, i, 0)),
        pl.BlockSpec((1, C), lambda i: (0, 0)),
        pl.BlockSpec((C, C), lambda i: (0, 0)),
        pl.BlockSpec((1, C), lambda i: (0, 0)),
    ],
    out_specs=pl.BlockSpec((BN, C), lambda i: (i, 0)),
    out_shape=jax.ShapeDtypeStruct((N, C), _f32),
)


@functools.partial(
    pl.kernel,
    out_type=[
        jax.ShapeDtypeStruct((EP,), _f32),
        jax.ShapeDtypeStruct((NC, NP), _f32),
        jax.ShapeDtypeStruct((EP,), _i32),
    ],
    mesh=_mesh,
    scratch_types=[
        pltpu.VMEM((CHW,), _i32),
        pltpu.VMEM((CHW,), _i32),
        pltpu.VMEM((CHW,), _i32),
        pltpu.VMEM((CHW,), _f32),
        pltpu.VMEM((CHW,), _i32),
        pltpu.VMEM((CH,), _i32),
        pltpu.VMEM((CH,), _i32),
        pltpu.VMEM((CH,), _f32),
        pltpu.VMEM((CH,), _f32),
        pltpu.VMEM((NP,), _f32),
        pltpu.VMEM((STR,), _f32),
        pltpu.VMEM((STR,), _f32),
        pltpu.VMEM_SHARED((NS, NP), _f32),
        pltpu.SemaphoreType.DMA,
        pltpu.SemaphoreType.DMA,
    ],
    compiler_params=_sc_params,
)
def _phase_a(qa_hbm, ka_hbm, src_hbm, dst_hbm, et_hbm,
             ev_out, den_out, pk_out,
             srcb, dstb, etb, evb, pkb, idxq, idxk, qsing, ksing,
             dloc, acc, tmp, stage, sem1, sem2):
    c = lax.axis_index("c")
    s = lax.axis_index("s")
    wid = s * NC + c
    base = wid * CHW
    pltpu.sync_copy(src_hbm.at[pl.ds(base, CHW)], srcb)
    pltpu.sync_copy(dst_hbm.at[pl.ds(base, CHW)], dstb)
    pltpu.sync_copy(et_hbm.at[pl.ds(base, CHW)], etb)
    zero16 = jnp.zeros((16,), _f32)

    def zb(i, _):
        dloc[pl.ds(i * 16, 16)] = zero16
        return 0

    lax.fori_loop(0, NP // 16, zb, 0, unroll=8)
    iota = lax.iota(_i32, 16)

    def chunk(ci, _):
        off = ci * CH
        for g in range(CH // 16):
            sl = pl.ds(g * 16, 16)
            gsl = pl.ds(off + g * 16, 16)
            etg = etb[gsl]
            dv = dstb[gsl]
            si = srcb[gsl] * R + etg
            idxq[sl] = dv * R + etg
            idxk[sl] = si
            pkb[gsl] = lax.shift_left(si, DBITS) | dv
        d1 = pltpu.async_copy(qa_hbm.at[idxq], qsing, sem1)
        d2 = pltpu.async_copy(ka_hbm.at[idxk], ksing, sem2)
        d1.wait()
        d2.wait()
        gbase = base + off
        for g in range(CH // 16):
            sl = pl.ds(g * 16, 16)
            gsl = pl.ds(off + g * 16, 16)
            a = qsing[sl] + ksing[sl]
            a = jnp.where(a > 0, a, a * NEG)
            ev = jnp.exp(a)
            eidx = gbase + g * 16 + iota
            ev = jnp.where(eidx < E, ev, 0.0)
            evb[gsl] = ev
            plsc.addupdate_scatter(dloc, [dstb[gsl]], ev)
        return 0

    lax.fori_loop(0, NCH, chunk, 0)
    pltpu.sync_copy(evb, ev_out.at[pl.ds(base, CHW)])
    pltpu.sync_copy(pkb, pk_out.at[pl.ds(base, CHW)])
    pltpu.sync_copy(dloc, stage.at[s])
    plsc.subcore_barrier()
    rb = s * STR
    pltpu.sync_copy(stage.at[0, pl.ds(rb, STR)], acc)

    def redj(j, _):
        pltpu.sync_copy(stage.at[j, pl.ds(rb, STR)], tmp)
        for g in range(STR // 16):
            sl = pl.ds(g * 16, 16)
            acc[sl] = acc[sl] + tmp[sl]
        return 0

    lax.fori_loop(1, NS, redj, 0)
    pltpu.sync_copy(acc, den_out.at[c, pl.ds(rb, STR)])


@functools.partial(
    pl.kernel,
    out_type=[
        jax.ShapeDtypeStruct((NC, NP, C), _f32),
        jax.ShapeDtypeStruct((EP,), _f32),
    ],
    mesh=_mesh,
    scratch_types=[
        [pltpu.VMEM((CB,), _i32)] * 2,
        [pltpu.VMEM((CB,), _f32)] * 2,
        [pltpu.VMEM((CB,), _f32)] * 2,
        [pltpu.VMEM((CB,), _i32)] * 2,
        [pltpu.VMEM((CB,), _i32)] * 2,
        [pltpu.VMEM((CB, C), _f32)] * 2,
        pltpu.VMEM((NP,), _f32),
        pltpu.VMEM((NP,), _f32),
        pltpu.VMEM_SHARED((NP, C), _f32),
        [pltpu.SemaphoreType.DMA] * 2,
        [pltpu.SemaphoreType.DMA] * 2,
        [pltpu.SemaphoreType.DMA] * 2,
        [pltpu.SemaphoreType.DMA] * 2,
        [pltpu.SemaphoreType.DMA] * 2,
    ],
    compiler_params=_sc_params,
)
def _phase_b(ao_hbm, ev_hbm, den_hbm, pk_hbm,
             part_out, al_out,
             pkc, evc, alc, idxb, didxb, rows, dfull, tmp, aggr,
             semP, semE, semG, semS, semA):
    c = lax.axis_index("c")
    s = lax.axis_index("s")
    wid = s * NC + c
    base = wid * CHW
    rb = s * STR
    zero16 = jnp.zeros((16,), _f32)

    def zrow(i, _):
        for j in range(C // 16):
            rows[0][i, pl.ds(j * 16, 16)] = zero16
        return 0

    lax.fori_loop(0, CB, zrow, 0)
    for p in range(STR // CB):
        pltpu.sync_copy(rows[0], aggr.at[pl.ds(rb + p * CB, CB)])

    pltpu.sync_copy(den_hbm.at[0], dfull)
    pltpu.sync_copy(den_hbm.at[1], tmp)

    def db(i, _):
        sl = pl.ds(i * 16, 16)
        dfull[sl] = dfull[sl] + tmp[sl]
        return 0

    lax.fori_loop(0, NP // 16, db, 0, unroll=8)
    plsc.subcore_barrier()

    def load_meta(ci, b):
        gb = base + ci * CB
        pltpu.async_copy(pk_hbm.at[pl.ds(gb, CB)], pkc[b], semP[b])
        pltpu.async_copy(ev_hbm.at[pl.ds(gb, CB)], evc[b], semE[b])

    def wait_meta_pk(ci, b):
        gb = base + ci * CB
        pltpu.make_async_copy(pk_hbm.at[pl.ds(gb, CB)], pkc[b], semP[b]).wait()

    def wait_meta_ev(ci, b):
        gb = base + ci * CB
        pltpu.make_async_copy(ev_hbm.at[pl.ds(gb, CB)], evc[b], semE[b]).wait()

    load_meta(0, 0)
    load_meta(1, 1)

    def wait_scatter(b):
        pltpu.make_async_copy(rows[b], aggr.at[pl.ds(rb, CB)], semS[b]).wait()

    def emit_front(cc, b, p):
        wait_meta_pk(cc, b)

        @pl.when(p > 0)
        def _():
            wait_scatter(b)

        for g in range(CB // 16):
            sl = pl.ds(g * 16, 16)
            pv = pkc[b][sl]
            idxb[b][sl] = lax.shift_right_logical(pv, DBITS)
            didxb[b][sl] = pv & DMASK
        return pltpu.async_copy(ao_hbm.at[idxb[b]], rows[b], semG[b])

    def emit_back(cc, b, p, gdma):
        wait_meta_ev(cc, b)
        for g in range(CB // 16):
            sl = pl.ds(g * 16, 16)
            den_g = plsc.load_gather(dfull, [didxb[b][sl]])
            alc[b][sl] = evc[b][sl] / (den_g + 1e-16)
        pltpu.sync_copy(alc[b], al_out.at[pl.ds(base + cc * CB, CB)])

        @pl.when(cc + 2 < NCB)
        def _():
            load_meta(cc + 2, b)

        gdma.wait()

        def eb(i, _):
            ri = jnp.zeros((16,), _i32) + i
            siv = plsc.load_gather(alc[b], [ri])
            for j in range(C // 16):
                sl = pl.ds(j * 16, 16)
                rows[b][i, sl] = rows[b][i, sl] * siv
            return 0

        lax.fori_loop(0, CB, eb, 0, unroll=4)
        pltpu.async_copy(rows[b], aggr.at[didxb[b]], semS[b], add=True)

    def pair(p, _):
        c0 = 2 * p
        c1 = c0 + 1
        g0 = emit_front(c0, 0, p)
        g1 = emit_front(c1, 1, p)
        emit_back(c0, 0, p, g0)
        emit_back(c1, 1, p, g1)
        return 0

    lax.fori_loop(0, NCB // 2, pair, 0)
    wait_scatter(0)
    wait_scatter(1)
    plsc.subcore_barrier()
    pltpu.sync_copy(aggr.at[pl.ds(rb, STR)], part_out.at[c, pl.ds(rb, STR)])


def kernel(x, edge_index, edge_type, w1, q1, k1, b1, w2, q2, k2, b2,
           lin_w, lin_b):
    src = edge_index[0]
    dst = edge_index[1]
    pad = EP - E
    srcp = jnp.pad(src, (0, pad))
    dstp = jnp.pad(dst, (0, pad))
    etp = jnp.pad(edge_type, (0, pad))

    wcat1 = w1.transpose(1, 0, 2).reshape(C, R * C)
    wcat2 = w2.transpose(1, 0, 2).reshape(C, R * C)
    wq1, wk1, wq2, wk2 = _wproj(w1, q1, k1, w2, q2, k2)
    wq1 = wq1.reshape(R, C)
    wk1 = wk1.reshape(R, C)
    wq2 = wq2.reshape(R, C)
    wk2 = wk2.reshape(R, C)

    ao1, qa1, ka1 = _tc1(x, wcat1, wq1, wk1)
    ev1, den1, pk = _phase_a(qa1.reshape(N * R), ka1.reshape(N * R),
                             srcp, dstp, etp)
    part1, _ = _phase_b(ao1.reshape(N * R, C), ev1, den1, pk)
    ao2, qa2, ka2 = _tcmid(part1, b1.reshape(1, C), wcat2, wq2, wk2)
    ev2, den2, _ = _phase_a(qa2.reshape(N * R), ka2.reshape(N * R),
                            srcp, dstp, etp)
    part2, al2 = _phase_b(ao2.reshape(N * R, C), ev2, den2, pk)
    out = _tcfin(part2, b2.reshape(1, C), lin_w.T, lin_b.reshape(1, C))
    alpha2 = al2[:E].reshape(E, 1)
    return out, (edge_index, alpha2)


# speedup vs baseline: 20.8364x; 1.1013x over previous
import functools

import jax
import jax.numpy as jnp
from jax import lax
from jax.experimental import pallas as pl
from jax.experimental.pallas import tpu as pltpu
from jax.experimental.pallas import tpu_sc as plsc

N = 10000
E = 320000
C = 128
R = 16
NEG = 0.2
NC = 2
NS = 16
NW = NC * NS
CH = 128
NCH = -(-E // (NW * CH))
CHW = NCH * CH
EP = CHW * NW
CB = 64
NCB = CHW // CB
NP = 10240
STR = NP // NS
DBITS = 14
DMASK = (1 << DBITS) - 1
BN = 1000

_mesh = plsc.VectorSubcoreMesh(core_axis_name="c", subcore_axis_name="s")
_f32 = jnp.float32
_i32 = jnp.int32
_sc_params = pltpu.CompilerParams(needs_layout_passes=False)


def _wproj_body(w1_ref, q1_ref, k1_ref, w2_ref, q2_ref, k2_ref,
                wq1_ref, wk1_ref, wq2_ref, wk2_ref):
    w1f = w1_ref[...].reshape(R * C, C)
    w2f = w2_ref[...].reshape(R * C, C)
    wq1_ref[...] = jnp.dot(w1f, q1_ref[...], preferred_element_type=_f32)
    wk1_ref[...] = jnp.dot(w1f, k1_ref[...], preferred_element_type=_f32)
    wq2_ref[...] = jnp.dot(w2f, q2_ref[...], preferred_element_type=_f32)
    wk2_ref[...] = jnp.dot(w2f, k2_ref[...], preferred_element_type=_f32)


_wproj = pl.pallas_call(
    _wproj_body,
    out_shape=[jax.ShapeDtypeStruct((R * C, 1), _f32)] * 4,
)


def _qk(xb, wq, wk):
    dn = (((1,), (1,)), ((), ()))
    qa = lax.dot_general(xb, wq, dn, preferred_element_type=_f32)
    ka = lax.dot_general(xb, wk, dn, preferred_element_type=_f32)
    return qa, ka


def _tcqk1_body(x_ref, wq_ref, wk_ref, qa_ref, ka_ref):
    qa_ref[...], ka_ref[...] = _qk(x_ref[...], wq_ref[...], wk_ref[...])


_qkspecs = dict(
    out_specs=[
        pl.BlockSpec((BN, R), lambda i: (i, 0)),
        pl.BlockSpec((BN, R), lambda i: (i, 0)),
    ],
    out_shape=[
        jax.ShapeDtypeStruct((N, R), _f32),
        jax.ShapeDtypeStruct((N, R), _f32),
    ],
)

_tcqk1 = pl.pallas_call(
    _tcqk1_body,
    grid=(N // BN,),
    in_specs=[
        pl.BlockSpec((BN, C), lambda i: (i, 0)),
        pl.BlockSpec((R, C), lambda i: (0, 0)),
        pl.BlockSpec((R, C), lambda i: (0, 0)),
    ],
    **_qkspecs,
)


def _tcao1_body(x_ref, wcat_ref, o_ref):
    o_ref[...] = jnp.dot(x_ref[...], wcat_ref[...],
                         preferred_element_type=_f32)


_tcao1 = pl.pallas_call(
    _tcao1_body,
    grid=(N // BN,),
    in_specs=[
        pl.BlockSpec((BN, C), lambda i: (i, 0)),
        pl.BlockSpec((C, R * C), lambda i: (0, 0)),
    ],
    out_specs=pl.BlockSpec((BN, R * C), lambda i: (i, 0)),
    out_shape=jax.ShapeDtypeStruct((N, R * C), _f32),
)


def _tcqk2_body(p_ref, b_ref, wq_ref, wk_ref, qa_ref, ka_ref):
    h = jnp.maximum(p_ref[0] + p_ref[1] + b_ref[...], 0.0)
    qa_ref[...], ka_ref[...] = _qk(h, wq_ref[...], wk_ref[...])


_tcqk2 = pl.pallas_call(
    _tcqk2_body,
    grid=(N // BN,),
    in_specs=[
        pl.BlockSpec((NC, BN, C), lambda i: (0, i, 0)),
        pl.BlockSpec((1, C), lambda i: (0, 0)),
        pl.BlockSpec((R, C), lambda i: (0, 0)),
        pl.BlockSpec((R, C), lambda i: (0, 0)),
    ],
    **_qkspecs,
)


def _tcao2_body(p_ref, b_ref, wcat_ref, o_ref):
    h = jnp.maximum(p_ref[0] + p_ref[1] + b_ref[...], 0.0)
    o_ref[...] = jnp.dot(h, wcat_ref[...], preferred_element_type=_f32)


_tcao2 = pl.pallas_call(
    _tcao2_body,
    grid=(N // BN,),
    in_specs=[
        pl.BlockSpec((NC, BN, C), lambda i: (0, i, 0)),
        pl.BlockSpec((1, C), lambda i: (0, 0)),
        pl.BlockSpec((C, R * C), lambda i: (0, 0)),
    ],
    out_specs=pl.BlockSpec((BN, R * C), lambda i: (i, 0)),
    out_shape=jax.ShapeDtypeStruct((N, R * C), _f32),
)


def _tcfin_body(p_ref, b_ref, linT_ref, lb_ref, o_ref):
    h = jnp.maximum(p_ref[0] + p_ref[1] + b_ref[...], 0.0)
    o_ref[...] = jnp.dot(h, linT_ref[...], preferred_element_type=_f32) + lb_ref[...]


_tcfin = pl.pallas_call(
    _tcfin_body,
    grid=(N // BN,),
    in_specs=[
        pl.BlockSpec((NC, BN, C), lambda i: (0, i, 0)),
        pl.BlockSpec((1, C), lambda i: (0, 0)),
        pl.BlockSpec((C, C), lambda i: (0, 0)),
        pl.BlockSpec((1, C), lambda i: (0, 0)),
    ],
    out_specs=pl.BlockSpec((BN, C), lambda i: (i, 0)),
    out_shape=jax.ShapeDtypeStruct((N, C), _f32),
)


@functools.partial(
    pl.kernel,
    out_type=[
        jax.ShapeDtypeStruct((EP,), _f32),
        jax.ShapeDtypeStruct((NC, NP), _f32),
        jax.ShapeDtypeStruct((EP,), _i32),
    ],
    mesh=_mesh,
    scratch_types=[
        pltpu.VMEM((CHW,), _i32),
        pltpu.VMEM((CHW,), _i32),
        pltpu.VMEM((CHW,), _i32),
        pltpu.VMEM((CHW,), _f32),
        pltpu.VMEM((CHW,), _i32),
        [pltpu.VMEM((CH,), _i32)] * 2,
        [pltpu.VMEM((CH,), _i32)] * 2,
        [pltpu.VMEM((CH,), _f32)] * 2,
        [pltpu.VMEM((CH,), _f32)] * 2,
        pltpu.VMEM((NP,), _f32),
        pltpu.VMEM((STR,), _f32),
        pltpu.VMEM((STR,), _f32),
        pltpu.VMEM_SHARED((NS, NP), _f32),
        [pltpu.SemaphoreType.DMA] * 2,
        [pltpu.SemaphoreType.DMA] * 2,
    ],
    compiler_params=_sc_params,
)
def _phase_a(qa_hbm, ka_hbm, src_hbm, dst_hbm, et_hbm,
             ev_out, den_out, pk_out,
             srcb, dstb, etb, evb, pkb, idxq, idxk, qsing, ksing,
             dloc, acc, tmp, stage, semq, semk):
    c = lax.axis_index("c")
    s = lax.axis_index("s")
    wid = s * NC + c
    base = wid * CHW
    pltpu.sync_copy(src_hbm.at[pl.ds(base, CHW)], srcb)
    pltpu.sync_copy(dst_hbm.at[pl.ds(base, CHW)], dstb)
    pltpu.sync_copy(et_hbm.at[pl.ds(base, CHW)], etb)
    zero16 = jnp.zeros((16,), _f32)

    def zb(i, _):
        dloc[pl.ds(i * 16, 16)] = zero16
        return 0

    lax.fori_loop(0, NP // 16, zb, 0, unroll=8)
    iota = lax.iota(_i32, 16)

    def afront(ci, b):
        off = ci * CH
        for g in range(CH // 16):
            sl = pl.ds(g * 16, 16)
            gsl = pl.ds(off + g * 16, 16)
            etg = etb[gsl]
            dv = dstb[gsl]
            si = srcb[gsl] * R + etg
            idxq[b][sl] = dv * R + etg
            idxk[b][sl] = si
            pkb[gsl] = lax.shift_left(si, DBITS) | dv
        d1 = pltpu.async_copy(qa_hbm.at[idxq[b]], qsing[b], semq[b])
        d2 = pltpu.async_copy(ka_hbm.at[idxk[b]], ksing[b], semk[b])
        return d1, d2

    def aback(ci, b, dmas):
        dmas[0].wait()
        dmas[1].wait()
        off = ci * CH
        gbase = base + off
        for g in range(CH // 16):
            sl = pl.ds(g * 16, 16)
            gsl = pl.ds(off + g * 16, 16)
            a = qsing[b][sl] + ksing[b][sl]
            a = jnp.where(a > 0, a, a * NEG)
            ev = jnp.exp(a)
            eidx = gbase + g * 16 + iota
            ev = jnp.where(eidx < E, ev, 0.0)
            evb[gsl] = ev
            plsc.addupdate_scatter(dloc, [dstb[gsl]], ev)

    def apair(p, _):
        c0 = 2 * p
        c1 = c0 + 1
        g0 = afront(c0, 0)
        g1 = afront(c1, 1)
        aback(c0, 0, g0)
        aback(c1, 1, g1)
        return 0

    lax.fori_loop(0, NCH // 2, apair, 0)
    gl = afront(NCH - 1, 0)
    aback(NCH - 1, 0, gl)
    pltpu.sync_copy(evb, ev_out.at[pl.ds(base, CHW)])
    pltpu.sync_copy(pkb, pk_out.at[pl.ds(base, CHW)])
    pltpu.sync_copy(dloc, stage.at[s])
    plsc.subcore_barrier()
    rb = s * STR
    pltpu.sync_copy(stage.at[0, pl.ds(rb, STR)], acc)

    def redj(j, _):
        pltpu.sync_copy(stage.at[j, pl.ds(rb, STR)], tmp)
        for g in range(STR // 16):
            sl = pl.ds(g * 16, 16)
            acc[sl] = acc[sl] + tmp[sl]
        return 0

    lax.fori_loop(1, NS, redj, 0)
    pltpu.sync_copy(acc, den_out.at[c, pl.ds(rb, STR)])


@functools.partial(
    pl.kernel,
    out_type=[
        jax.ShapeDtypeStruct((NC, NP, C), _f32),
        jax.ShapeDtypeStruct((EP,), _f32),
    ],
    mesh=_mesh,
    scratch_types=[
        [pltpu.VMEM((CB,), _i32)] * 2,
        [pltpu.VMEM((CB,), _f32)] * 2,
        [pltpu.VMEM((CB,), _f32)] * 2,
        [pltpu.VMEM((CB,), _i32)] * 2,
        [pltpu.VMEM((CB,), _i32)] * 2,
        [pltpu.VMEM((CB, C), _f32)] * 2,
        pltpu.VMEM((NP,), _f32),
        pltpu.VMEM((NP,), _f32),
        pltpu.VMEM_SHARED((NP, C), _f32),
        [pltpu.SemaphoreType.DMA] * 2,
        [pltpu.SemaphoreType.DMA] * 2,
        [pltpu.SemaphoreType.DMA] * 2,
        [pltpu.SemaphoreType.DMA] * 2,
        [pltpu.SemaphoreType.DMA] * 2,
    ],
    compiler_params=_sc_params,
)
def _phase_b(ao_hbm, ev_hbm, den_hbm, pk_hbm,
             part_out, al_out,
             pkc, evc, alc, idxb, didxb, rows, dfull, tmp, aggr,
             semP, semE, semG, semS, semA):
    c = lax.axis_index("c")
    s = lax.axis_index("s")
    wid = s * NC + c
    base = wid * CHW
    rb = s * STR
    zero16 = jnp.zeros((16,), _f32)

    def zrow(i, _):
        for j in range(C // 16):
            rows[0][i, pl.ds(j * 16, 16)] = zero16
        return 0

    lax.fori_loop(0, CB, zrow, 0)
    for p in range(STR // CB):
        pltpu.sync_copy(rows[0], aggr.at[pl.ds(rb + p * CB, CB)])

    pltpu.sync_copy(den_hbm.at[0], dfull)
    pltpu.sync_copy(den_hbm.at[1], tmp)

    def db(i, _):
        sl = pl.ds(i * 16, 16)
        dfull[sl] = dfull[sl] + tmp[sl]
        return 0

    lax.fori_loop(0, NP // 16, db, 0, unroll=8)
    plsc.subcore_barrier()

    def load_meta(ci, b):
        gb = base + ci * CB
        pltpu.async_copy(pk_hbm.at[pl.ds(gb, CB)], pkc[b], semP[b])
        pltpu.async_copy(ev_hbm.at[pl.ds(gb, CB)], evc[b], semE[b])

    def wait_meta_pk(ci, b):
        gb = base + ci * CB
        pltpu.make_async_copy(pk_hbm.at[pl.ds(gb, CB)], pkc[b], semP[b]).wait()

    def wait_meta_ev(ci, b):
        gb = base + ci * CB
        pltpu.make_async_copy(ev_hbm.at[pl.ds(gb, CB)], evc[b], semE[b]).wait()

    load_meta(0, 0)
    load_meta(1, 1)

    def wait_scatter(b):
        pltpu.make_async_copy(rows[b], aggr.at[pl.ds(rb, CB)], semS[b]).wait()

    def emit_front(cc, b, p):
        wait_meta_pk(cc, b)

        @pl.when(p > 0)
        def _():
            wait_scatter(b)

        for g in range(CB // 16):
            sl = pl.ds(g * 16, 16)
            pv = pkc[b][sl]
            idxb[b][sl] = lax.shift_right_logical(pv, DBITS)
            didxb[b][sl] = pv & DMASK
        return pltpu.async_copy(ao_hbm.at[idxb[b]], rows[b], semG[b])

    def wait_alpha(b):
        pltpu.make_async_copy(alc[b], al_out.at[pl.ds(base, CB)],
                              semA[b]).wait()

    def emit_back(cc, b, p, gdma):
        wait_meta_ev(cc, b)

        @pl.when(p > 0)
        def _():
            wait_alpha(b)

        for g in range(CB // 16):
            sl = pl.ds(g * 16, 16)
            den_g = plsc.load_gather(dfull, [didxb[b][sl]])
            alc[b][sl] = evc[b][sl] / (den_g + 1e-16)
        pltpu.async_copy(alc[b], al_out.at[pl.ds(base + cc * CB, CB)],
                         semA[b])

        @pl.when(cc + 2 < NCB)
        def _():
            load_meta(cc + 2, b)

        gdma.wait()

        def eb(i, _):
            ri = jnp.zeros((16,), _i32) + i
            siv = plsc.load_gather(alc[b], [ri])
            for j in range(C // 16):
                sl = pl.ds(j * 16, 16)
                rows[b][i, sl] = rows[b][i, sl] * siv
            return 0

        lax.fori_loop(0, CB, eb, 0, unroll=4)
        pltpu.async_copy(rows[b], aggr.at[didxb[b]], semS[b], add=True)

    def pair(p, _):
        c0 = 2 * p
        c1 = c0 + 1
        g0 = emit_front(c0, 0, p)
        g1 = emit_front(c1, 1, p)
        emit_back(c0, 0, p, g0)
        emit_back(c1, 1, p, g1)
        return 0

    lax.fori_loop(0, NCB // 2, pair, 0)
    wait_scatter(0)
    wait_scatter(1)
    wait_alpha(0)
    wait_alpha(1)
    plsc.subcore_barrier()
    pltpu.sync_copy(aggr.at[pl.ds(rb, STR)], part_out.at[c, pl.ds(rb, STR)])


def kernel(x, edge_index, edge_type, w1, q1, k1, b1, w2, q2, k2, b2,
           lin_w, lin_b):
    src = edge_index[0]
    dst = edge_index[1]
    pad = EP - E
    srcp = jnp.pad(src, (0, pad))
    dstp = jnp.pad(dst, (0, pad))
    etp = jnp.pad(edge_type, (0, pad))

    wcat1 = w1.transpose(1, 0, 2).reshape(C, R * C)
    wcat2 = w2.transpose(1, 0, 2).reshape(C, R * C)
    wq1, wk1, wq2, wk2 = _wproj(w1, q1, k1, w2, q2, k2)
    wq1 = wq1.reshape(R, C)
    wk1 = wk1.reshape(R, C)
    wq2 = wq2.reshape(R, C)
    wk2 = wk2.reshape(R, C)

    qa1, ka1 = _tcqk1(x, wq1, wk1)
    ao1 = _tcao1(x, wcat1)
    ev1, den1, pk = _phase_a(qa1.reshape(N * R), ka1.reshape(N * R),
                             srcp, dstp, etp)
    part1, _ = _phase_b(ao1.reshape(N * R, C), ev1, den1, pk)
    b1r = b1.reshape(1, C)
    qa2, ka2 = _tcqk2(part1, b1r, wq2, wk2)
    ao2 = _tcao2(part1, b1r, wcat2)
    ev2, den2, _ = _phase_a(qa2.reshape(N * R), ka2.reshape(N * R),
                            srcp, dstp, etp)
    part2, al2 = _phase_b(ao2.reshape(N * R, C), ev2, den2, pk)
    out = _tcfin(part2, b2.reshape(1, C), lin_w.T, lin_b.reshape(1, C))
    alpha2 = al2[:E].reshape(E, 1)
    return out, (edge_index, alpha2)


# speedup vs baseline: 21.3704x; 1.0256x over previous
import functools

import jax
import jax.numpy as jnp
from jax import lax
from jax.experimental import pallas as pl
from jax.experimental.pallas import tpu as pltpu
from jax.experimental.pallas import tpu_sc as plsc

N = 10000
E = 320000
C = 128
R = 16
NEG = 0.2
NC = 2
NS = 16
NW = NC * NS
CH = 128
NCH = -(-E // (NW * CH))
CHW = NCH * CH
EP = CHW * NW
CB = 128
NCB = CHW // CB
NP = 10240
STR = NP // NS
DBITS = 14
DMASK = (1 << DBITS) - 1
BN = 1000

_mesh = plsc.VectorSubcoreMesh(core_axis_name="c", subcore_axis_name="s")
_f32 = jnp.float32
_i32 = jnp.int32
_sc_params = pltpu.CompilerParams(needs_layout_passes=False)


def _wproj_body(w1_ref, q1_ref, k1_ref, w2_ref, q2_ref, k2_ref,
                wq1_ref, wk1_ref, wq2_ref, wk2_ref):
    w1f = w1_ref[...].reshape(R * C, C)
    w2f = w2_ref[...].reshape(R * C, C)
    wq1_ref[...] = jnp.dot(w1f, q1_ref[...], preferred_element_type=_f32)
    wk1_ref[...] = jnp.dot(w1f, k1_ref[...], preferred_element_type=_f32)
    wq2_ref[...] = jnp.dot(w2f, q2_ref[...], preferred_element_type=_f32)
    wk2_ref[...] = jnp.dot(w2f, k2_ref[...], preferred_element_type=_f32)


_wproj = pl.pallas_call(
    _wproj_body,
    out_shape=[jax.ShapeDtypeStruct((R * C, 1), _f32)] * 4,
)


def _qk(xb, wq, wk):
    dn = (((1,), (1,)), ((), ()))
    qa = lax.dot_general(xb, wq, dn, preferred_element_type=_f32)
    ka = lax.dot_general(xb, wk, dn, preferred_element_type=_f32)
    return qa, ka


def _tcqk1_body(x_ref, wq_ref, wk_ref, qa_ref, ka_ref):
    qa_ref[...], ka_ref[...] = _qk(x_ref[...], wq_ref[...], wk_ref[...])


_qkspecs = dict(
    out_specs=[
        pl.BlockSpec((BN, R), lambda i: (i, 0)),
        pl.BlockSpec((BN, R), lambda i: (i, 0)),
    ],
    out_shape=[
        jax.ShapeDtypeStruct((N, R), _f32),
        jax.ShapeDtypeStruct((N, R), _f32),
    ],
)

_tcqk1 = pl.pallas_call(
    _tcqk1_body,
    grid=(N // BN,),
    in_specs=[
        pl.BlockSpec((BN, C), lambda i: (i, 0)),
        pl.BlockSpec((R, C), lambda i: (0, 0)),
        pl.BlockSpec((R, C), lambda i: (0, 0)),
    ],
    **_qkspecs,
)


def _tcao1_body(x_ref, wcat_ref, o_ref):
    o_ref[...] = jnp.dot(x_ref[...], wcat_ref[...],
                         preferred_element_type=_f32)


_tcao1 = pl.pallas_call(
    _tcao1_body,
    grid=(N // BN,),
    in_specs=[
        pl.BlockSpec((BN, C), lambda i: (i, 0)),
        pl.BlockSpec((C, R * C), lambda i: (0, 0)),
    ],
    out_specs=pl.BlockSpec((BN, R * C), lambda i: (i, 0)),
    out_shape=jax.ShapeDtypeStruct((N, R * C), _f32),
)


def _tcqk2_body(p_ref, b_ref, wq_ref, wk_ref, qa_ref, ka_ref):
    h = jnp.maximum(p_ref[0] + p_ref[1] + b_ref[...], 0.0)
    qa_ref[...], ka_ref[...] = _qk(h, wq_ref[...], wk_ref[...])


_tcqk2 = pl.pallas_call(
    _tcqk2_body,
    grid=(N // BN,),
    in_specs=[
        pl.BlockSpec((NC, BN, C), lambda i: (0, i, 0)),
        pl.BlockSpec((1, C), lambda i: (0, 0)),
        pl.BlockSpec((R, C), lambda i: (0, 0)),
        pl.BlockSpec((R, C), lambda i: (0, 0)),
    ],
    **_qkspecs,
)


def _tcao2_body(p_ref, b_ref, wcat_ref, o_ref):
    h = jnp.maximum(p_ref[0] + p_ref[1] + b_ref[...], 0.0)
    o_ref[...] = jnp.dot(h, wcat_ref[...], preferred_element_type=_f32)


_tcao2 = pl.pallas_call(
    _tcao2_body,
    grid=(N // BN,),
    in_specs=[
        pl.BlockSpec((NC, BN, C), lambda i: (0, i, 0)),
        pl.BlockSpec((1, C), lambda i: (0, 0)),
        pl.BlockSpec((C, R * C), lambda i: (0, 0)),
    ],
    out_specs=pl.BlockSpec((BN, R * C), lambda i: (i, 0)),
    out_shape=jax.ShapeDtypeStruct((N, R * C), _f32),
)


def _tcfin_body(p_ref, b_ref, linT_ref, lb_ref, o_ref):
    h = jnp.maximum(p_ref[0] + p_ref[1] + b_ref[...], 0.0)
    o_ref[...] = jnp.dot(h, linT_ref[...], preferred_element_type=_f32) + lb_ref[...]


_tcfin = pl.pallas_call(
    _tcfin_body,
    grid=(N // BN,),
    in_specs=[
        pl.BlockSpec((NC, BN, C), lambda i: (0, i, 0)),
        pl.BlockSpec((1, C), lambda i: (0, 0)),
        pl.BlockSpec((C, C), lambda i: (0, 0)),
        pl.BlockSpec((1, C), lambda i: (0, 0)),
    ],
    out_specs=pl.BlockSpec((BN, C), lambda i: (i, 0)),
    out_shape=jax.ShapeDtypeStruct((N, C), _f32),
)


@functools.partial(
    pl.kernel,
    out_type=[
        jax.ShapeDtypeStruct((EP,), _f32),
        jax.ShapeDtypeStruct((NC, NP), _f32),
        jax.ShapeDtypeStruct((EP,), _i32),
    ],
    mesh=_mesh,
    scratch_types=[
        pltpu.VMEM((CHW,), _i32),
        pltpu.VMEM((CHW,), _i32),
        pltpu.VMEM((CHW,), _i32),
        pltpu.VMEM((CHW,), _f32),
        pltpu.VMEM((CHW,), _i32),
        [pltpu.VMEM((CH,), _i32)] * 2,
        [pltpu.VMEM((CH,), _i32)] * 2,
        [pltpu.VMEM((CH,), _f32)] * 2,
        [pltpu.VMEM((CH,), _f32)] * 2,
        pltpu.VMEM((NP,), _f32),
        pltpu.VMEM((STR,), _f32),
        pltpu.VMEM((STR,), _f32),
        pltpu.VMEM_SHARED((NS, NP), _f32),
        [pltpu.SemaphoreType.DMA] * 2,
        [pltpu.SemaphoreType.DMA] * 2,
    ],
    compiler_params=_sc_params,
)
def _phase_a(qa_hbm, ka_hbm, src_hbm, dst_hbm, et_hbm,
             ev_out, den_out, pk_out,
             srcb, dstb, etb, evb, pkb, idxq, idxk, qsing, ksing,
             dloc, acc, tmp, stage, semq, semk):
    c = lax.axis_index("c")
    s = lax.axis_index("s")
    wid = s * NC + c
    base = wid * CHW
    pltpu.sync_copy(src_hbm.at[pl.ds(base, CHW)], srcb)
    pltpu.sync_copy(dst_hbm.at[pl.ds(base, CHW)], dstb)
    pltpu.sync_copy(et_hbm.at[pl.ds(base, CHW)], etb)
    zero16 = jnp.zeros((16,), _f32)

    def zb(i, _):
        dloc[pl.ds(i * 16, 16)] = zero16
        return 0

    lax.fori_loop(0, NP // 16, zb, 0, unroll=8)
    iota = lax.iota(_i32, 16)

    def afront(ci, b):
        off = ci * CH
        for g in range(CH // 16):
            sl = pl.ds(g * 16, 16)
            gsl = pl.ds(off + g * 16, 16)
            etg = etb[gsl]
            dv = dstb[gsl]
            si = srcb[gsl] * R + etg
            idxq[b][sl] = dv * R + etg
            idxk[b][sl] = si
            pkb[gsl] = lax.shift_left(si, DBITS) | dv
        d1 = pltpu.async_copy(qa_hbm.at[idxq[b]], qsing[b], semq[b])
        d2 = pltpu.async_copy(ka_hbm.at[idxk[b]], ksing[b], semk[b])
        return d1, d2

    def aback(ci, b, dmas):
        dmas[0].wait()
        dmas[1].wait()
        off = ci * CH
        gbase = base + off
        for g in range(CH // 16):
            sl = pl.ds(g * 16, 16)
            gsl = pl.ds(off + g * 16, 16)
            a = qsing[b][sl] + ksing[b][sl]
            a = jnp.where(a > 0, a, a * NEG)
            ev = jnp.exp(a)
            eidx = gbase + g * 16 + iota
            ev = jnp.where(eidx < E, ev, 0.0)
            evb[gsl] = ev
            plsc.addupdate_scatter(dloc, [dstb[gsl]], ev)

    def apair(p, _):
        c0 = 2 * p
        c1 = c0 + 1
        g0 = afront(c0, 0)
        g1 = afront(c1, 1)
        aback(c0, 0, g0)
        aback(c1, 1, g1)
        return 0

    lax.fori_loop(0, NCH // 2, apair, 0)
    gl = afront(NCH - 1, 0)
    aback(NCH - 1, 0, gl)
    pltpu.sync_copy(evb, ev_out.at[pl.ds(base, CHW)])
    pltpu.sync_copy(pkb, pk_out.at[pl.ds(base, CHW)])
    pltpu.sync_copy(dloc, stage.at[s])
    plsc.subcore_barrier()
    rb = s * STR
    pltpu.sync_copy(stage.at[0, pl.ds(rb, STR)], acc)

    def redj(j, _):
        pltpu.sync_copy(stage.at[j, pl.ds(rb, STR)], tmp)
        for g in range(STR // 16):
            sl = pl.ds(g * 16, 16)
            acc[sl] = acc[sl] + tmp[sl]
        return 0

    lax.fori_loop(1, NS, redj, 0)
    pltpu.sync_copy(acc, den_out.at[c, pl.ds(rb, STR)])


@functools.partial(
    pl.kernel,
    out_type=[
        jax.ShapeDtypeStruct((NC, NP, C), _f32),
        jax.ShapeDtypeStruct((EP,), _f32),
    ],
    mesh=_mesh,
    scratch_types=[
        [pltpu.VMEM((CB,), _i32)] * 2,
        [pltpu.VMEM((CB,), _f32)] * 2,
        [pltpu.VMEM((CB,), _f32)] * 2,
        [pltpu.VMEM((CB,), _i32)] * 2,
        [pltpu.VMEM((CB,), _i32)] * 2,
        [pltpu.VMEM((CB, C), _f32)] * 2,
        pltpu.VMEM((NP,), _f32),
        pltpu.VMEM_SHARED((NP, C), _f32),
        [pltpu.SemaphoreType.DMA] * 2,
        [pltpu.SemaphoreType.DMA] * 2,
        [pltpu.SemaphoreType.DMA] * 2,
        [pltpu.SemaphoreType.DMA] * 2,
        [pltpu.SemaphoreType.DMA] * 2,
    ],
    compiler_params=_sc_params,
)
def _phase_b(ao_hbm, ev_hbm, den_hbm, pk_hbm,
             part_out, al_out,
             pkc, evc, alc, idxb, didxb, rows, dfull, aggr,
             semP, semE, semG, semS, semA):
    c = lax.axis_index("c")
    s = lax.axis_index("s")
    wid = s * NC + c
    base = wid * CHW
    rb = s * STR
    zero16 = jnp.zeros((16,), _f32)

    DR = NP // C
    pltpu.sync_copy(den_hbm.at[0], rows[0].at[pl.ds(0, DR)])
    pltpu.sync_copy(den_hbm.at[1], rows[1].at[pl.ds(0, DR)])

    def db(p, _):
        for j in range(C // 16):
            sl = pl.ds(j * 16, 16)
            dfull[pl.ds(p * C + j * 16, 16)] = rows[0][p, sl] + rows[1][p, sl]
        return 0

    lax.fori_loop(0, DR, db, 0, unroll=4)

    def zrow(i, _):
        for j in range(C // 16):
            rows[0][i, pl.ds(j * 16, 16)] = zero16
        return 0

    lax.fori_loop(0, CB, zrow, 0)
    for p in range(STR // CB):
        pltpu.sync_copy(rows[0], aggr.at[pl.ds(rb + p * CB, CB)])
    plsc.subcore_barrier()

    def load_meta(ci, b):
        gb = base + ci * CB
        pltpu.async_copy(pk_hbm.at[pl.ds(gb, CB)], pkc[b], semP[b])
        pltpu.async_copy(ev_hbm.at[pl.ds(gb, CB)], evc[b], semE[b])

    def wait_meta_pk(ci, b):
        gb = base + ci * CB
        pltpu.make_async_copy(pk_hbm.at[pl.ds(gb, CB)], pkc[b], semP[b]).wait()

    def wait_meta_ev(ci, b):
        gb = base + ci * CB
        pltpu.make_async_copy(ev_hbm.at[pl.ds(gb, CB)], evc[b], semE[b]).wait()

    load_meta(0, 0)
    load_meta(1, 1)

    def wait_scatter(b):
        pltpu.make_async_copy(rows[b], aggr.at[pl.ds(rb, CB)], semS[b]).wait()

    def emit_front(cc, b, p):
        wait_meta_pk(cc, b)

        @pl.when(p > 0)
        def _():
            wait_scatter(b)

        for g in range(CB // 16):
            sl = pl.ds(g * 16, 16)
            pv = pkc[b][sl]
            idxb[b][sl] = lax.shift_right_logical(pv, DBITS)
            didxb[b][sl] = pv & DMASK
        return pltpu.async_copy(ao_hbm.at[idxb[b]], rows[b], semG[b])

    def wait_alpha(b):
        pltpu.make_async_copy(alc[b], al_out.at[pl.ds(base, CB)],
                              semA[b]).wait()

    def emit_back(cc, b, p, gdma):
        wait_meta_ev(cc, b)

        @pl.when(p > 0)
        def _():
            wait_alpha(b)

        for g in range(CB // 16):
            sl = pl.ds(g * 16, 16)
            den_g = plsc.load_gather(dfull, [didxb[b][sl]])
            alc[b][sl] = evc[b][sl] / (den_g + 1e-16)
        pltpu.async_copy(alc[b], al_out.at[pl.ds(base + cc * CB, CB)],
                         semA[b])

        @pl.when(cc + 2 < NCB)
        def _():
            load_meta(cc + 2, b)

        gdma.wait()

        def eb(i, _):
            ri = jnp.zeros((16,), _i32) + i
            siv = plsc.load_gather(alc[b], [ri])
            for j in range(C // 16):
                sl = pl.ds(j * 16, 16)
                rows[b][i, sl] = rows[b][i, sl] * siv
            return 0

        lax.fori_loop(0, CB, eb, 0, unroll=8)
        pltpu.async_copy(rows[b], aggr.at[didxb[b]], semS[b], add=True)

    def pair(p, _):
        c0 = 2 * p
        c1 = c0 + 1
        g0 = emit_front(c0, 0, p)
        g1 = emit_front(c1, 1, p)
        emit_back(c0, 0, p, g0)
        emit_back(c1, 1, p, g1)
        return 0

    lax.fori_loop(0, NCB // 2, pair, 0)
    gl = emit_front(NCB - 1, 0, NCB // 2)
    emit_back(NCB - 1, 0, NCB // 2, gl)
    wait_scatter(0)
    wait_scatter(1)
    wait_alpha(0)
    wait_alpha(1)
    plsc.subcore_barrier()
    pltpu.sync_copy(aggr.at[pl.ds(rb, STR)], part_out.at[c, pl.ds(rb, STR)])


def kernel(x, edge_index, edge_type, w1, q1, k1, b1, w2, q2, k2, b2,
           lin_w, lin_b):
    src = edge_index[0]
    dst = edge_index[1]
    pad = EP - E
    srcp = jnp.pad(src, (0, pad))
    dstp = jnp.pad(dst, (0, pad))
    etp = jnp.pad(edge_type, (0, pad))

    wcat1 = w1.transpose(1, 0, 2).reshape(C, R * C)
    wcat2 = w2.transpose(1, 0, 2).reshape(C, R * C)
    wq1, wk1, wq2, wk2 = _wproj(w1, q1, k1, w2, q2, k2)
    wq1 = wq1.reshape(R, C)
    wk1 = wk1.reshape(R, C)
    wq2 = wq2.reshape(R, C)
    wk2 = wk2.reshape(R, C)

    qa1, ka1 = _tcqk1(x, wq1, wk1)
    ao1 = _tcao1(x, wcat1)
    ev1, den1, pk = _phase_a(qa1.reshape(N * R), ka1.reshape(N * R),
                             srcp, dstp, etp)
    part1, _ = _phase_b(ao1.reshape(N * R, C), ev1,
                        den1.reshape(NC, NP // C, C), pk)
    b1r = b1.reshape(1, C)
    qa2, ka2 = _tcqk2(part1, b1r, wq2, wk2)
    ao2 = _tcao2(part1, b1r, wcat2)
    ev2, den2, _ = _phase_a(qa2.reshape(N * R), ka2.reshape(N * R),
                            srcp, dstp, etp)
    part2, al2 = _phase_b(ao2.reshape(N * R, C), ev2,
                          den2.reshape(NC, NP // C, C), pk)
    out = _tcfin(part2, b2.reshape(1, C), lin_w.T, lin_b.reshape(1, C))
    alpha2 = al2[:E].reshape(E, 1)
    return out, (edge_index, alpha2)


# speedup vs baseline: 21.8963x; 1.0246x over previous
import functools

import jax
import jax.numpy as jnp
from jax import lax
from jax.experimental import pallas as pl
from jax.experimental.pallas import tpu as pltpu
from jax.experimental.pallas import tpu_sc as plsc

N = 10000
E = 320000
C = 128
R = 16
NEG = 0.2
NC = 2
NS = 16
NW = NC * NS
CH = 128
NCH = -(-E // (NW * CH))
CHW = NCH * CH
EP = CHW * NW
CB = 128
NCB = CHW // CB
NP = 10240
STR = NP // NS
DBITS = 14
DMASK = (1 << DBITS) - 1
BN = 1000

_mesh = plsc.VectorSubcoreMesh(core_axis_name="c", subcore_axis_name="s")
_f32 = jnp.float32
_i32 = jnp.int32
_sc_params = pltpu.CompilerParams(needs_layout_passes=False)


def _wproj_body(w1_ref, q1_ref, k1_ref, w2_ref, q2_ref, k2_ref,
                wq1_ref, wk1_ref, wq2_ref, wk2_ref):
    w1f = w1_ref[...].reshape(R * C, C)
    w2f = w2_ref[...].reshape(R * C, C)
    wq1_ref[...] = jnp.dot(w1f, q1_ref[...], preferred_element_type=_f32)
    wk1_ref[...] = jnp.dot(w1f, k1_ref[...], preferred_element_type=_f32)
    wq2_ref[...] = jnp.dot(w2f, q2_ref[...], preferred_element_type=_f32)
    wk2_ref[...] = jnp.dot(w2f, k2_ref[...], preferred_element_type=_f32)


_wproj = pl.pallas_call(
    _wproj_body,
    out_shape=[jax.ShapeDtypeStruct((R * C, 1), _f32)] * 4,
)


def _qk(xb, wq, wk):
    dn = (((1,), (1,)), ((), ()))
    qa = lax.dot_general(xb, wq, dn, preferred_element_type=_f32)
    ka = lax.dot_general(xb, wk, dn, preferred_element_type=_f32)
    return qa, ka


def _tcqk1_body(x_ref, wq_ref, wk_ref, qa_ref, ka_ref):
    qa_ref[...], ka_ref[...] = _qk(x_ref[...], wq_ref[...], wk_ref[...])


_qkspecs = dict(
    out_specs=[
        pl.BlockSpec((BN, R), lambda i: (i, 0)),
        pl.BlockSpec((BN, R), lambda i: (i, 0)),
    ],
    out_shape=[
        jax.ShapeDtypeStruct((N, R), _f32),
        jax.ShapeDtypeStruct((N, R), _f32),
    ],
)

_tcqk1 = pl.pallas_call(
    _tcqk1_body,
    grid=(N // BN,),
    in_specs=[
        pl.BlockSpec((BN, C), lambda i: (i, 0)),
        pl.BlockSpec((R, C), lambda i: (0, 0)),
        pl.BlockSpec((R, C), lambda i: (0, 0)),
    ],
    **_qkspecs,
)


def _tcao1_body(x_ref, w_ref, o_ref):
    o_ref[0] = jnp.dot(x_ref[...], w_ref[0], preferred_element_type=_f32)


_tcao1 = pl.pallas_call(
    _tcao1_body,
    grid=(N // BN, R),
    in_specs=[
        pl.BlockSpec((BN, C), lambda i, r: (i, 0)),
        pl.BlockSpec((1, C, C), lambda i, r: (r, 0, 0)),
    ],
    out_specs=pl.BlockSpec((1, BN, C), lambda i, r: (r, i, 0)),
    out_shape=jax.ShapeDtypeStruct((R, N, C), _f32),
)


def _tcqk2_body(p_ref, b_ref, wq_ref, wk_ref, qa_ref, ka_ref):
    h = jnp.maximum(p_ref[0] + p_ref[1] + b_ref[...], 0.0)
    qa_ref[...], ka_ref[...] = _qk(h, wq_ref[...], wk_ref[...])


_tcqk2 = pl.pallas_call(
    _tcqk2_body,
    grid=(N // BN,),
    in_specs=[
        pl.BlockSpec((NC, BN, C), lambda i: (0, i, 0)),
        pl.BlockSpec((1, C), lambda i: (0, 0)),
        pl.BlockSpec((R, C), lambda i: (0, 0)),
        pl.BlockSpec((R, C), lambda i: (0, 0)),
    ],
    **_qkspecs,
)


def _tcao2_body(p_ref, b_ref, w_ref, o_ref):
    h = jnp.maximum(p_ref[0] + p_ref[1] + b_ref[...], 0.0)
    o_ref[0] = jnp.dot(h, w_ref[0], preferred_element_type=_f32)


_tcao2 = pl.pallas_call(
    _tcao2_body,
    grid=(N // BN, R),
    in_specs=[
        pl.BlockSpec((NC, BN, C), lambda i, r: (0, i, 0)),
        pl.BlockSpec((1, C), lambda i, r: (0, 0)),
        pl.BlockSpec((1, C, C), lambda i, r: (r, 0, 0)),
    ],
    out_specs=pl.BlockSpec((1, BN, C), lambda i, r: (r, i, 0)),
    out_shape=jax.ShapeDtypeStruct((R, N, C), _f32),
)


def _tcfin_body(p_ref, b_ref, linT_ref, lb_ref, o_ref):
    h = jnp.maximum(p_ref[0] + p_ref[1] + b_ref[...], 0.0)
    o_ref[...] = jnp.dot(h, linT_ref[...], preferred_element_type=_f32) + lb_ref[...]


_tcfin = pl.pallas_call(
    _tcfin_body,
    grid=(N // BN,),
    in_specs=[
        pl.BlockSpec((NC, BN, C), lambda i: (0, i, 0)),
        pl.BlockSpec((1, C), lambda i: (0, 0)),
        pl.BlockSpec((C, C), lambda i: (0, 0)),
        pl.BlockSpec((1, C), lambda i: (0, 0)),
    ],
    out_specs=pl.BlockSpec((BN, C), lambda i: (i, 0)),
    out_shape=jax.ShapeDtypeStruct((N, C), _f32),
)


@functools.partial(
    pl.kernel,
    out_type=[
        jax.ShapeDtypeStruct((EP,), _f32),
        jax.ShapeDtypeStruct((NC, NP), _f32),
        jax.ShapeDtypeStruct((EP,), _i32),
    ],
    mesh=_mesh,
    scratch_types=[
        pltpu.VMEM((CHW,), _i32),
        pltpu.VMEM((CHW,), _i32),
        pltpu.VMEM((CHW,), _i32),
        pltpu.VMEM((CHW,), _f32),
        pltpu.VMEM((CHW,), _i32),
        [pltpu.VMEM((CH,), _i32)] * 2,
        [pltpu.VMEM((CH,), _i32)] * 2,
        [pltpu.VMEM((CH,), _f32)] * 2,
        [pltpu.VMEM((CH,), _f32)] * 2,
        pltpu.VMEM((NP,), _f32),
        pltpu.VMEM((STR,), _f32),
        pltpu.VMEM((STR,), _f32),
        pltpu.VMEM_SHARED((NS, NP), _f32),
        [pltpu.SemaphoreType.DMA] * 2,
        [pltpu.SemaphoreType.DMA] * 2,
    ],
    compiler_params=_sc_params,
)
def _phase_a(qa_hbm, ka_hbm, src_hbm, dst_hbm, et_hbm,
             ev_out, den_out, pk_out,
             srcb, dstb, etb, evb, pkb, idxq, idxk, qsing, ksing,
             dloc, acc, tmp, stage, semq, semk):
    c = lax.axis_index("c")
    s = lax.axis_index("s")
    wid = s * NC + c
    base = wid * CHW
    pltpu.sync_copy(src_hbm.at[pl.ds(base, CHW)], srcb)
    pltpu.sync_copy(dst_hbm.at[pl.ds(base, CHW)], dstb)
    pltpu.sync_copy(et_hbm.at[pl.ds(base, CHW)], etb)
    zero16 = jnp.zeros((16,), _f32)

    def zb(i, _):
        dloc[pl.ds(i * 16, 16)] = zero16
        return 0

    lax.fori_loop(0, NP // 16, zb, 0, unroll=8)
    iota = lax.iota(_i32, 16)

    def afront(ci, b):
        off = ci * CH
        for g in range(CH // 16):
            sl = pl.ds(g * 16, 16)
            gsl = pl.ds(off + g * 16, 16)
            etg = etb[gsl]
            dv = dstb[gsl]
            si = etg * N + srcb[gsl]
            idxq[b][sl] = dv * R + etg
            idxk[b][sl] = srcb[gsl] * R + etg
            pkb[gsl] = lax.shift_left(si, DBITS) | dv
        d1 = pltpu.async_copy(qa_hbm.at[idxq[b]], qsing[b], semq[b])
        d2 = pltpu.async_copy(ka_hbm.at[idxk[b]], ksing[b], semk[b])
        return d1, d2

    def aback(ci, b, dmas):
        dmas[0].wait()
        dmas[1].wait()
        off = ci * CH
        gbase = base + off
        for g in range(CH // 16):
            sl = pl.ds(g * 16, 16)
            gsl = pl.ds(off + g * 16, 16)
            a = qsing[b][sl] + ksing[b][sl]
            a = jnp.where(a > 0, a, a * NEG)
            ev = jnp.exp(a)
            eidx = gbase + g * 16 + iota
            ev = jnp.where(eidx < E, ev, 0.0)
            evb[gsl] = ev
            plsc.addupdate_scatter(dloc, [dstb[gsl]], ev)

    def apair(p, _):
        c0 = 2 * p
        c1 = c0 + 1
        g0 = afront(c0, 0)
        g1 = afront(c1, 1)
        aback(c0, 0, g0)
        aback(c1, 1, g1)
        return 0

    lax.fori_loop(0, NCH // 2, apair, 0)
    gl = afront(NCH - 1, 0)
    aback(NCH - 1, 0, gl)
    pltpu.sync_copy(evb, ev_out.at[pl.ds(base, CHW)])
    pltpu.sync_copy(pkb, pk_out.at[pl.ds(base, CHW)])
    pltpu.sync_copy(dloc, stage.at[s])
    plsc.subcore_barrier()
    rb = s * STR
    pltpu.sync_copy(stage.at[0, pl.ds(rb, STR)], acc)

    def redj(j, _):
        pltpu.sync_copy(stage.at[j, pl.ds(rb, STR)], tmp)
        for g in range(STR // 16):
            sl = pl.ds(g * 16, 16)
            acc[sl] = acc[sl] + tmp[sl]
        return 0

    lax.fori_loop(1, NS, redj, 0)
    pltpu.sync_copy(acc, den_out.at[c, pl.ds(rb, STR)])


@functools.partial(
    pl.kernel,
    out_type=[
        jax.ShapeDtypeStruct((NC, NP, C), _f32),
        jax.ShapeDtypeStruct((EP,), _f32),
    ],
    mesh=_mesh,
    scratch_types=[
        [pltpu.VMEM((CB,), _i32)] * 2,
        [pltpu.VMEM((CB,), _f32)] * 2,
        [pltpu.VMEM((CB,), _f32)] * 2,
        [pltpu.VMEM((CB,), _i32)] * 2,
        [pltpu.VMEM((CB,), _i32)] * 2,
        [pltpu.VMEM((CB, C), _f32)] * 2,
        pltpu.VMEM((NP,), _f32),
        pltpu.VMEM_SHARED((NP, C), _f32),
        [pltpu.SemaphoreType.DMA] * 2,
        [pltpu.SemaphoreType.DMA] * 2,
        [pltpu.SemaphoreType.DMA] * 2,
        [pltpu.SemaphoreType.DMA] * 2,
        [pltpu.SemaphoreType.DMA] * 2,
    ],
    compiler_params=_sc_params,
)
def _phase_b(ao_hbm, ev_hbm, den_hbm, pk_hbm,
             part_out, al_out,
             pkc, evc, alc, idxb, didxb, rows, dfull, aggr,
             semP, semE, semG, semS, semA):
    c = lax.axis_index("c")
    s = lax.axis_index("s")
    wid = s * NC + c
    base = wid * CHW
    rb = s * STR
    zero16 = jnp.zeros((16,), _f32)

    DR = NP // C
    pltpu.sync_copy(den_hbm.at[0], rows[0].at[pl.ds(0, DR)])
    pltpu.sync_copy(den_hbm.at[1], rows[1].at[pl.ds(0, DR)])

    def db(p, _):
        for j in range(C // 16):
            sl = pl.ds(j * 16, 16)
            dfull[pl.ds(p * C + j * 16, 16)] = rows[0][p, sl] + rows[1][p, sl]
        return 0

    lax.fori_loop(0, DR, db, 0, unroll=4)

    def zrow(i, _):
        for j in range(C // 16):
            rows[0][i, pl.ds(j * 16, 16)] = zero16
        return 0

    lax.fori_loop(0, CB, zrow, 0)
    for p in range(STR // CB):
        pltpu.sync_copy(rows[0], aggr.at[pl.ds(rb + p * CB, CB)])
    plsc.subcore_barrier()

    def load_meta(ci, b):
        gb = base + ci * CB
        pltpu.async_copy(pk_hbm.at[pl.ds(gb, CB)], pkc[b], semP[b])
        pltpu.async_copy(ev_hbm.at[pl.ds(gb, CB)], evc[b], semE[b])

    def wait_meta_pk(ci, b):
        gb = base + ci * CB
        pltpu.make_async_copy(pk_hbm.at[pl.ds(gb, CB)], pkc[b], semP[b]).wait()

    def wait_meta_ev(ci, b):
        gb = base + ci * CB
        pltpu.make_async_copy(ev_hbm.at[pl.ds(gb, CB)], evc[b], semE[b]).wait()

    load_meta(0, 0)
    load_meta(1, 1)

    def wait_scatter(b):
        pltpu.make_async_copy(rows[b], aggr.at[pl.ds(rb, CB)], semS[b]).wait()

    def emit_front(cc, b, p):
        wait_meta_pk(cc, b)

        @pl.when(p > 0)
        def _():
            wait_scatter(b)

        for g in range(CB // 16):
            sl = pl.ds(g * 16, 16)
            pv = pkc[b][sl]
            idxb[b][sl] = lax.shift_right_logical(pv, DBITS)
            didxb[b][sl] = pv & DMASK
        return pltpu.async_copy(ao_hbm.at[idxb[b]], rows[b], semG[b])

    def wait_alpha(b):
        pltpu.make_async_copy(alc[b], al_out.at[pl.ds(base, CB)],
                              semA[b]).wait()

    def emit_back(cc, b, p, gdma):
        wait_meta_ev(cc, b)

        @pl.when(p > 0)
        def _():
            wait_alpha(b)

        for g in range(CB // 16):
            sl = pl.ds(g * 16, 16)
            den_g = plsc.load_gather(dfull, [didxb[b][sl]])
            alc[b][sl] = evc[b][sl] / (den_g + 1e-16)
        pltpu.async_copy(alc[b], al_out.at[pl.ds(base + cc * CB, CB)],
                         semA[b])

        @pl.when(cc + 2 < NCB)
        def _():
            load_meta(cc + 2, b)

        gdma.wait()

        def eb(i, _):
            ri = jnp.zeros((16,), _i32) + i
            siv = plsc.load_gather(alc[b], [ri])
            for j in range(C // 16):
                sl = pl.ds(j * 16, 16)
                rows[b][i, sl] = rows[b][i, sl] * siv
            return 0

        lax.fori_loop(0, CB, eb, 0, unroll=8)
        pltpu.async_copy(rows[b], aggr.at[didxb[b]], semS[b], add=True)

    def pair(p, _):
        c0 = 2 * p
        c1 = c0 + 1
        g0 = emit_front(c0, 0, p)
        g1 = emit_front(c1, 1, p)
        emit_back(c0, 0, p, g0)
        emit_back(c1, 1, p, g1)
        return 0

    lax.fori_loop(0, NCB // 2, pair, 0)
    gl = emit_front(NCB - 1, 0, NCB // 2)
    emit_back(NCB - 1, 0, NCB // 2, gl)
    wait_scatter(0)
    wait_scatter(1)
    wait_alpha(0)
    wait_alpha(1)
    plsc.subcore_barrier()
    pltpu.sync_copy(aggr.at[pl.ds(rb, STR)], part_out.at[c, pl.ds(rb, STR)])


def kernel(x, edge_index, edge_type, w1, q1, k1, b1, w2, q2, k2, b2,
           lin_w, lin_b):
    src = edge_index[0]
    dst = edge_index[1]
    pad = EP - E
    srcp = jnp.pad(src, (0, pad))
    dstp = jnp.pad(dst, (0, pad))
    etp = jnp.pad(edge_type, (0, pad))

    wq1, wk1, wq2, wk2 = _wproj(w1, q1, k1, w2, q2, k2)
    wq1 = wq1.reshape(R, C)
    wk1 = wk1.reshape(R, C)
    wq2 = wq2.reshape(R, C)
    wk2 = wk2.reshape(R, C)

    qa1, ka1 = _tcqk1(x, wq1, wk1)
    ao1 = _tcao1(x, w1)
    ev1, den1, pk = _phase_a(qa1.reshape(N * R), ka1.reshape(N * R),
                             srcp, dstp, etp)
    part1, _ = _phase_b(ao1.reshape(R * N, C), ev1,
                        den1.reshape(NC, NP // C, C), pk)
    b1r = b1.reshape(1, C)
    qa2, ka2 = _tcqk2(part1, b1r, wq2, wk2)
    ao2 = _tcao2(part1, b1r, w2)
    ev2, den2, _ = _phase_a(qa2.reshape(N * R), ka2.reshape(N * R),
                            srcp, dstp, etp)
    part2, al2 = _phase_b(ao2.reshape(R * N, C), ev2,
                          den2.reshape(NC, NP // C, C), pk)
    out = _tcfin(part2, b2.reshape(1, C), lin_w.T, lin_b.reshape(1, C))
    alpha2 = al2[:E].reshape(E, 1)
    return out, (edge_index, alpha2)
